# baseline (speedup 1.0000x reference)
import functools

import jax
import jax.numpy as jnp
from jax import lax
from jax.experimental import pallas as pl
from jax.experimental.pallas import tpu as pltpu
from jax.experimental.pallas import tpu_sc as plsc

N = 50000
E = 200000
G = 10000
H = 64

NC, NS = 2, 16
CH = 128

S_NODE = 25088
NP = 2 * S_NODE
ACC_N = 25600
EP = 200704
K_E = EP // NS // CH
K_G = 2 * EP // (NC * NS) // CH
S_G = 5120
GP = 2 * S_G
ACC_G = 5248
NSP = 51200
K_B = NSP // NS // CH

BE = 1024
BN = 1024

@functools.cache
def _sc_mesh():
  return plsc.VectorSubcoreMesh(
      core_axis_name="c", subcore_axis_name="s", num_cores=NC, num_subcores=NS)


def _sc_gather(table, idx4, d, k, dtype=jnp.float32):
  rows_pt = k * CH

  def body(table_h, idx_h, out_h, idx_v, buf_a, buf_b, sem_a, sem_b):
    cid = lax.axis_index("c")
    sid = lax.axis_index("s")
    base = (cid * NS + sid) * rows_pt
    pltpu.sync_copy(idx_h.at[cid, sid], idx_v)

    def step(j, carry):
      @pl.when(lax.rem(j, 2) == 0)
      def _():
        pltpu.async_copy(table_h.at[idx_v.at[j]], buf_a, sem_a).wait()
        pltpu.sync_copy(buf_a, out_h.at[pl.ds(base + j * CH, CH)])

      @pl.when(lax.rem(j, 2) == 1)
      def _():
        pltpu.async_copy(table_h.at[idx_v.at[j]], buf_b, sem_b).wait()
        pltpu.sync_copy(buf_b, out_h.at[pl.ds(base + j * CH, CH)])
      return carry

    lax.fori_loop(0, k, step, 0, unroll=False)

  f = pl.kernel(
      body,
      out_type=jax.ShapeDtypeStruct((NC * NS * rows_pt, d), dtype),
      mesh=_sc_mesh(),
      compiler_params=pltpu.CompilerParams(use_tc_tiling_on_sc=False),
      scratch_types=[
          pltpu.VMEM((k, CH), jnp.int32),
          pltpu.VMEM((CH, d), dtype),
          pltpu.VMEM((CH, d), dtype),
          pltpu.SemaphoreType.DMA,
          pltpu.SemaphoreType.DMA,
      ],
  )
  return f(table, idx4)


def _sc_scatter_add(vals, idx4, zeros, d, k, s_half, acc_rows):
  z_pt = acc_rows // NS
  o_pt = s_half // NS
  nfull, rem = divmod(z_pt, CH)

  def body(vals_h, idx_h, zeros_h, out_h, idx_v, vbuf, acc):
    cid = lax.axis_index("c")
    sid = lax.axis_index("s")
    pltpu.sync_copy(idx_h.at[cid, sid], idx_v)

    zb = sid * z_pt
    for t in range(nfull):
      pltpu.sync_copy(zeros_h, acc.at[pl.ds(zb + t * CH, CH)])
    if rem:
      pltpu.sync_copy(zeros_h.at[pl.ds(0, rem)],
                      acc.at[pl.ds(zb + nfull * CH, rem)])
    plsc.subcore_barrier()

    tb = sid * (k * CH)

    def step(j, carry):
      pltpu.sync_copy(vals_h.at[pl.ds(tb + j * CH, CH)], vbuf)
      pltpu.sync_copy(vbuf, acc.at[idx_v.at[j]], add=True)
      return carry

    lax.fori_loop(0, k, step, 0, unroll=False)
    plsc.subcore_barrier()

    ob = sid * o_pt
    pltpu.sync_copy(acc.at[pl.ds(ob, o_pt)],
                    out_h.at[pl.ds(cid * s_half + ob, o_pt)])

  f = pl.kernel(
      body,
      out_type=jax.ShapeDtypeStruct((2 * s_half, d), jnp.float32),
      mesh=_sc_mesh(),
      compiler_params=pltpu.CompilerParams(use_tc_tiling_on_sc=False),
      scratch_types=[
          pltpu.VMEM((k, CH), jnp.int32),
          pltpu.VMEM((CH, d), jnp.float32),
          pltpu.VMEM_SHARED((acc_rows, d), jnp.float32),
      ],
  )
  return f(vals, idx4, zeros)


def _silu(x):
  return x * jax.nn.sigmoid(x)


def _sh4(r):
  n2 = jnp.sum(r * r, axis=1, keepdims=True)
  unit = r / jnp.clip(jnp.sqrt(n2), 1e-8, None)
  y0 = jnp.full((r.shape[0], 1), 0.28209479177387814, dtype=r.dtype)
  return jnp.concatenate([y0, 0.4886025119029199 * unit], axis=1)


def _tc_preproc(gpre):
  nb = EP // BE

  def kfn(gd_ref, gs_ref, ea_ref, es_ref):
    gd = gd_ref[...]
    gs = gs_ref[...]
    rel = gs[:, 0:3] - gd[:, 0:3]
    n2 = jnp.sum(rel * rel, axis=1, keepdims=True)
    dist = jnp.sqrt(n2 + 1e-12)
    ea4 = _sh4(rel)
    pc = gs[:, 3:4] * gd[:, 3:4]
    one = jnp.ones((BE, 1), jnp.float32)
    zero = jnp.zeros((BE, 4), jnp.float32)
    ea_ref[...] = jnp.concatenate([ea4, one] + [zero] * 2 + [zero[:, :3]],
                                  axis=1)
    es_ref[...] = jnp.concatenate([ea4, ea4 * dist, ea4 * pc, zero], axis=1)

  return pl.pallas_call(
      kfn,
      grid=(nb,),
      in_specs=[
          pl.BlockSpec((BE, 16), lambda i: (i, 0)),
          pl.BlockSpec((BE, 16), lambda i: (i + nb, 0)),
      ],
      out_specs=[
          pl.BlockSpec((BE, 16), lambda i: (i, 0)),
          pl.BlockSpec((BE, 16), lambda i: (i, 0)),
      ],
      out_shape=[
          jax.ShapeDtypeStruct((EP, 16), jnp.float32),
          jax.ShapeDtypeStruct((EP, 16), jnp.float32),
      ],
  )(gpre, gpre)


def _tc_embed(nf, mp, na, w, b):
  def kfn(nf_ref, mp_ref, na_ref, w_ref, b_ref, x0_ref, xb_ref, nat_ref):
    nf = nf_ref[...]
    pos = nf[:, 0:3]
    vel = nf[:, 5:8]
    v2 = jnp.sum(vel * vel, axis=1, keepdims=True)
    vel_abs = jnp.sqrt(v2 + 1e-12)
    vel_emb = _sh4(vel)
    na_v = na_ref[...]
    nattr = na_v[:, 0:4] / jnp.clip(na_v[:, 4:5], 1.0, None) + vel_emb
    mp_v = mp_ref[...]
    mean = mp_v[:, 0:3] / jnp.clip(mp_v[:, 4:5], 1.0, None)
    feat = jnp.concatenate(
        [pos - mean, vel, vel_abs, jnp.zeros((BN, 1), jnp.float32)], axis=1)
    acc = jnp.zeros((BN, H), jnp.float32) + b_ref[...]
    for a in range(4):
      acc = acc + nattr[:, a:a + 1] * jnp.dot(
          feat, w_ref[a], preferred_element_type=jnp.float32)
    x0_ref[...] = acc
    xb_ref[...] = acc.astype(jnp.bfloat16)
    nat_ref[...] = nattr

  nb = NP // BN
  return pl.pallas_call(
      kfn,
      grid=(nb,),
      in_specs=[
          pl.BlockSpec((BN, 16), lambda i: (i, 0)),
          pl.BlockSpec((BN, 16), lambda i: (i, 0)),
          pl.BlockSpec((BN, 16), lambda i: (i, 0)),
          pl.BlockSpec((4, 8, H), lambda i: (0, 0, 0)),
          pl.BlockSpec((1, H), lambda i: (0, 0)),
      ],
      out_specs=[
          pl.BlockSpec((BN, H), lambda i: (i, 0)),
          pl.BlockSpec((BN, H), lambda i: (i, 0)),
          pl.BlockSpec((BN, 4), lambda i: (i, 0)),
      ],
      out_shape=[
          jax.ShapeDtypeStruct((NP, H), jnp.float32),
          jax.ShapeDtypeStruct((NP, H), jnp.bfloat16),
          jax.ShapeDtypeStruct((NP, 4), jnp.float32),
      ],
  )(nf, mp, na, w, b)


def _tc_edge(g, escal, w1c, wdp, b1, w2c, b2):
  nb = EP // BE

  def kfn(xi_ref, xj_ref, es_ref, w1_ref, wdp_ref, b1_ref, w2_ref, b2_ref,
          out_ref):
    xij = jnp.concatenate([xi_ref[...], xj_ref[...]], axis=1)
    es = es_ref[...]
    t = jnp.dot(xij, w1_ref[...], preferred_element_type=jnp.float32)
    acc = b1_ref[...] + jnp.dot(es[:, 4:12], wdp_ref[...],
                                preferred_element_type=jnp.float32)
    for a in range(4):
      acc = acc + es[:, a:a + 1] * t[:, a * H:(a + 1) * H]
    m1 = _silu(acc).astype(jnp.bfloat16)
    t2 = jnp.dot(m1, w2_ref[...], preferred_element_type=jnp.float32)
    acc2 = jnp.zeros((BE, H), jnp.float32) + b2_ref[...]
    for a in range(4):
      acc2 = acc2 + es[:, a:a + 1] * t2[:, a * H:(a + 1) * H]
    out_ref[...] = _silu(acc2)

  return pl.pallas_call(
      kfn,
      grid=(nb,),
      in_specs=[
          pl.BlockSpec((BE, H), lambda i: (i, 0)),
          pl.BlockSpec((BE, H), lambda i: (i + nb, 0)),
          pl.BlockSpec((BE, 16), lambda i: (i, 0)),
          pl.BlockSpec((2 * H, 4 * H), lambda i: (0, 0)),
          pl.BlockSpec((8, H), lambda i: (0, 0)),
          pl.BlockSpec((1, H), lambda i: (0, 0)),
          pl.BlockSpec((H, 4 * H), lambda i: (0, 0)),
          pl.BlockSpec((1, H), lambda i: (0, 0)),
      ],
      out_specs=pl.BlockSpec((BE, H), lambda i: (i, 0)),
      out_shape=jax.ShapeDtypeStruct((EP, H), jnp.float32),
  )(g, g, escal, w1c, wdp, b1, w2c, b2)


def _tc_node(x, agg, nat, wu1c, b1, wu2c, b2):
  def kfn(x_ref, agg_ref, nat_ref, wu1_ref, b1_ref, wu2_ref,
          b2_ref, out_ref, outb_ref):
    x_v = x_ref[...]
    nat = nat_ref[...]
    xcat = jnp.concatenate([x_v, agg_ref[...]], axis=1).astype(jnp.bfloat16)
    t = jnp.dot(xcat, wu1_ref[...], preferred_element_type=jnp.float32)
    acc = jnp.zeros((BN, H), jnp.float32) + b1_ref[...]
    for a in range(4):
      acc = acc + nat[:, a:a + 1] * t[:, a * H:(a + 1) * H]
    u = _silu(acc).astype(jnp.bfloat16)
    t2 = jnp.dot(u, wu2_ref[...], preferred_element_type=jnp.float32)
    acc2 = jnp.zeros((BN, H), jnp.float32) + b2_ref[...]
    for a in range(4):
      acc2 = acc2 + nat[:, a:a + 1] * t2[:, a * H:(a + 1) * H]
    out = x_v + acc2
    out_ref[...] = out
    outb_ref[...] = out.astype(jnp.bfloat16)

  nb = NP // BN
  return pl.pallas_call(
      kfn,
      grid=(nb,),
      in_specs=[
          pl.BlockSpec((BN, H), lambda i: (i, 0)),
          pl.BlockSpec((BN, H), lambda i: (i, 0)),
          pl.BlockSpec((BN, 4), lambda i: (i, 0)),
          pl.BlockSpec((2 * H, 4 * H), lambda i: (0, 0)),
          pl.BlockSpec((1, H), lambda i: (0, 0)),
          pl.BlockSpec((H, 4 * H), lambda i: (0, 0)),
          pl.BlockSpec((1, H), lambda i: (0, 0)),
      ],
      out_specs=[
          pl.BlockSpec((BN, H), lambda i: (i, 0)),
          pl.BlockSpec((BN, H), lambda i: (i, 0)),
      ],
      out_shape=[
          jax.ShapeDtypeStruct((NP, H), jnp.float32),
          jax.ShapeDtypeStruct((NP, H), jnp.bfloat16),
      ],
  )(x, agg, nat, wu1c, b1, wu2c, b2)


def _tc_output(x, nat, nf, wo1, bo1, wo2, bo2):
  def kfn(x_ref, nat_ref, nf_ref, wo1_ref, bo1_ref, wo2_ref, bo2_ref,
          out_ref):
    x_v = x_ref[...]
    nat = nat_ref[...]
    acc = jnp.zeros((BN, H), jnp.float32) + bo1_ref[...]
    for a in range(4):
      acc = acc + nat[:, a:a + 1] * jnp.dot(
          x_v, wo1_ref[a], preferred_element_type=jnp.float32)
    u = _silu(acc)
    acc2 = jnp.zeros((BN, 128), jnp.float32) + bo2_ref[...]
    for a in range(4):
      acc2 = acc2 + nat[:, a:a + 1] * jnp.dot(
          u, wo2_ref[a], preferred_element_type=jnp.float32)
    pos = nf_ref[...][:, 0:3]
    out_ref[...] = acc2 + jnp.concatenate(
        [pos, jnp.zeros((BN, 125), jnp.float32)], axis=1)

  nb = NP // BN
  return pl.pallas_call(
      kfn,
      grid=(nb,),
      in_specs=[
          pl.BlockSpec((BN, H), lambda i: (i, 0)),
          pl.BlockSpec((BN, 4), lambda i: (i, 0)),
          pl.BlockSpec((BN, 16), lambda i: (i, 0)),
          pl.BlockSpec((4, H, H), lambda i: (0, 0, 0)),
          pl.BlockSpec((1, H), lambda i: (0, 0)),
          pl.BlockSpec((4, H, 128), lambda i: (0, 0, 0)),
          pl.BlockSpec((1, 128), lambda i: (0, 0)),
      ],
      out_specs=pl.BlockSpec((BN, 128), lambda i: (i, 0)),
      out_shape=jax.ShapeDtypeStruct((NP, 128), jnp.float32),
  )(x, nat, nf, wo1, bo1, wo2, bo2)


def _tp_weights(p):
  return p['W'].transpose(1, 0, 2), p['b'][None, :]


@jax.jit
def _run(pos, vel, charges, params, edge_index, batch):
  i32 = jnp.int32
  src = edge_index[0].astype(i32)
  dst = edge_index[1].astype(i32)
  batch = batch.astype(i32)

  nf = jnp.zeros((NP, 16), jnp.float32)
  nf = nf.at[:N, 0:3].set(pos)
  nf = nf.at[:N, 3].set(charges[:, 0])
  nf = nf.at[:N, 4].set(1.0)
  nf = nf.at[:N, 5:8].set(vel)

  pad_e = EP - E
  dst_g = jnp.concatenate([dst, jnp.zeros((pad_e,), i32)])
  src_g = jnp.concatenate([src, jnp.zeros((pad_e,), i32)])
  gidx = jnp.concatenate([dst_g, src_g]).reshape(NC, NS, K_G, CH)

  dst_p = jnp.concatenate([dst, jnp.full((pad_e,), 2 * S_NODE, i32)])
  s_lo = jnp.where(dst_p < S_NODE, dst_p, S_NODE)
  s_hi = jnp.where(dst_p >= S_NODE, dst_p - S_NODE, S_NODE)
  sidx = jnp.stack([s_lo, s_hi]).reshape(NC, NS, K_E, CH)

  pad_n = NSP - N
  bat_p = jnp.concatenate([batch, jnp.full((pad_n,), 2 * S_G, i32)])
  b_lo = jnp.where(bat_p < S_G, bat_p, S_G)
  b_hi = jnp.where(bat_p >= S_G, bat_p - S_G, S_G)
  bidx = jnp.stack([b_lo, b_hi]).reshape(NC, NS, K_B, CH)

  z16 = jnp.zeros((CH, 16), jnp.float32)
  z64 = jnp.zeros((CH, 64), jnp.float32)

  gpre = _sc_gather(nf, gidx, 16, K_G)
  ea16, escal = _tc_preproc(gpre)
  na = _sc_scatter_add(ea16, sidx, z16, 16, K_E, S_NODE, ACC_N)
  nf_sc = jnp.zeros((NSP, 16), jnp.float32).at[:NP].set(nf)
  mg = _sc_scatter_add(nf_sc, bidx, z16, 16, K_B, S_G, ACC_G)
  mp = jnp.zeros((NP, 16), jnp.float32).at[:N].set(
      jnp.repeat(mg[:G], 5, axis=0))

  w_emb, b_emb = _tp_weights(params['emb'])
  w_emb = jnp.pad(w_emb, ((0, 0), (0, 1), (0, 0)))
  x, xb, nat = _tc_embed(nf, mp, na, w_emb, b_emb)

  bf16 = jnp.bfloat16

  def _cat_w(w4):
    return w4.transpose(1, 0, 2).reshape(w4.shape[1], 4 * H).astype(bf16)

  for lp in params['layers']:
    w1, b1 = _tp_weights(lp['m1'])
    w1c = _cat_w(jnp.concatenate([w1[:, :H], w1[:, H:2 * H]], axis=1))
    wdp = jnp.concatenate([w1[:, 2 * H], w1[:, 2 * H + 1]], axis=0)
    w2, b2 = _tp_weights(lp['m2'])
    wu1, bu1 = _tp_weights(lp['u1'])
    wu2, bu2 = _tp_weights(lp['u2'])

    g = _sc_gather(xb, gidx, H, K_G, bf16)
    m2 = _tc_edge(g, escal, w1c, wdp, b1, _cat_w(w2), b2)
    agg = _sc_scatter_add(m2, sidx, z64, H, K_E, S_NODE, ACC_N)
    x, xb = _tc_node(x, agg, nat, _cat_w(wu1), bu1, _cat_w(wu2), bu2)

  wo1, bo1 = _tp_weights(params['o1'])
  wo2, bo2 = _tp_weights(params['o2'])
  wo2 = jnp.pad(wo2, ((0, 0), (0, 0), (0, 125)))
  bo2 = jnp.pad(bo2, ((0, 0), (0, 125)))
  out = _tc_output(x, nat, nf, wo1, bo1, wo2, bo2)
  return out[:N, :3]


def kernel(pos, vel, charges, params, edge_index, batch):
  return _run(pos, vel, charges, params, edge_index, batch)


# speedup vs baseline: 1.1326x; 1.1326x over previous
import functools

import jax
import jax.numpy as jnp
from jax import lax
from jax.experimental import pallas as pl
from jax.experimental.pallas import tpu as pltpu
from jax.experimental.pallas import tpu_sc as plsc

N = 50000
E = 200000
G = 10000
H = 64

NC, NS = 2, 16
CH = 128

S_NODE = 25600
NP = 2 * S_NODE
ACC_N = 25728
EP = 200704
K_E = EP // NS // CH
K_G = 2 * EP // (NC * NS) // CH
S_G = 5120
GP = 2 * S_G
ACC_G = 5248
K_B = NP // NS // CH

BE = 1024
BN = 1024

@functools.cache
def _sc_mesh():
  return plsc.VectorSubcoreMesh(
      core_axis_name="c", subcore_axis_name="s", num_cores=NC, num_subcores=NS)


def _sc_gather(table, idx4, d, k, dtype=jnp.float32):
  rows_pt = k * CH

  def body(table_h, idx_h, out_h, idx_v, buf_a, buf_b, sem_a, sem_b):
    cid = lax.axis_index("c")
    sid = lax.axis_index("s")
    base = (cid * NS + sid) * rows_pt
    pltpu.sync_copy(idx_h.at[cid, sid], idx_v)

    def step(j, carry):
      @pl.when(lax.rem(j, 2) == 0)
      def _():
        pltpu.async_copy(table_h.at[idx_v.at[j]], buf_a, sem_a).wait()
        pltpu.sync_copy(buf_a, out_h.at[pl.ds(base + j * CH, CH)])

      @pl.when(lax.rem(j, 2) == 1)
      def _():
        pltpu.async_copy(table_h.at[idx_v.at[j]], buf_b, sem_b).wait()
        pltpu.sync_copy(buf_b, out_h.at[pl.ds(base + j * CH, CH)])
      return carry

    lax.fori_loop(0, k, step, 0, unroll=False)

  f = pl.kernel(
      body,
      out_type=jax.ShapeDtypeStruct((NC * NS * rows_pt, d), dtype),
      mesh=_sc_mesh(),
      compiler_params=pltpu.CompilerParams(use_tc_tiling_on_sc=False),
      scratch_types=[
          pltpu.VMEM((k, CH), jnp.int32),
          pltpu.VMEM((CH, d), dtype),
          pltpu.VMEM((CH, d), dtype),
          pltpu.SemaphoreType.DMA,
          pltpu.SemaphoreType.DMA,
      ],
  )
  return f(table, idx4)


def _sc_scatter_add(vals, idx4, zeros, d, k, s_half, acc_rows):
  z_pt = acc_rows // NS
  o_pt = s_half // NS
  nfull, rem = divmod(z_pt, CH)

  def body(vals_h, idx_h, zeros_h, out_h, idx_v, vbuf, acc):
    cid = lax.axis_index("c")
    sid = lax.axis_index("s")
    pltpu.sync_copy(idx_h.at[cid, sid], idx_v)

    zb = sid * z_pt
    for t in range(nfull):
      pltpu.sync_copy(zeros_h, acc.at[pl.ds(zb + t * CH, CH)])
    if rem:
      pltpu.sync_copy(zeros_h.at[pl.ds(0, rem)],
                      acc.at[pl.ds(zb + nfull * CH, rem)])
    plsc.subcore_barrier()

    tb = sid * (k * CH)

    def step(j, carry):
      pltpu.sync_copy(vals_h.at[pl.ds(tb + j * CH, CH)], vbuf)
      pltpu.sync_copy(vbuf, acc.at[idx_v.at[j]], add=True)
      return carry

    lax.fori_loop(0, k, step, 0, unroll=False)
    plsc.subcore_barrier()

    ob = sid * o_pt
    pltpu.sync_copy(acc.at[pl.ds(ob, o_pt)],
                    out_h.at[pl.ds(cid * s_half + ob, o_pt)])

  f = pl.kernel(
      body,
      out_type=jax.ShapeDtypeStruct((2 * s_half, d), jnp.float32),
      mesh=_sc_mesh(),
      compiler_params=pltpu.CompilerParams(use_tc_tiling_on_sc=False),
      scratch_types=[
          pltpu.VMEM((k, CH), jnp.int32),
          pltpu.VMEM((CH, d), jnp.float32),
          pltpu.VMEM_SHARED((acc_rows, d), jnp.float32),
      ],
  )
  return f(vals, idx4, zeros)


def _silu(x):
  return x * jax.nn.sigmoid(x)


def _sh4(r):
  n2 = jnp.sum(r * r, axis=1, keepdims=True)
  unit = r / jnp.clip(jnp.sqrt(n2), 1e-8, None)
  y0 = jnp.full((r.shape[0], 1), 0.28209479177387814, dtype=r.dtype)
  return jnp.concatenate([y0, 0.4886025119029199 * unit], axis=1)


def _tc_nf(pos, vel, charges):
  BR = 2000

  def kfn(p_ref, v_ref, c_ref, out_ref):
    one = jnp.ones((BR, 1), jnp.float32)
    zero = jnp.zeros((BR, 8), jnp.float32)
    out_ref[...] = jnp.concatenate(
        [p_ref[...], c_ref[...], one, v_ref[...], zero], axis=1)

  return pl.pallas_call(
      kfn,
      grid=(N // BR,),
      in_specs=[
          pl.BlockSpec((BR, 3), lambda i: (i, 0)),
          pl.BlockSpec((BR, 3), lambda i: (i, 0)),
          pl.BlockSpec((BR, 1), lambda i: (i, 0)),
      ],
      out_specs=pl.BlockSpec((BR, 16), lambda i: (i, 0)),
      out_shape=jax.ShapeDtypeStruct((NP, 16), jnp.float32),
  )(pos, vel, charges)


def _tc_remap(idxf, n_valid, s_half):
  R = idxf.shape[0]
  nb = 4 if R % 32 == 0 else 1
  BR = R // nb

  def kfn(i_ref, lo_ref, hi_ref):
    i = pl.program_id(0)
    v = i_ref[...]
    row = jax.lax.broadcasted_iota(jnp.int32, (BR, CH), 0) + i * BR
    col = jax.lax.broadcasted_iota(jnp.int32, (BR, CH), 1)
    valid = row * CH + col < n_valid
    lo_ref[...] = jnp.where(valid & (v < s_half), v, s_half)
    hi_ref[...] = jnp.where(valid & (v >= s_half), v - s_half, s_half)

  return pl.pallas_call(
      kfn,
      grid=(nb,),
      in_specs=[pl.BlockSpec((BR, CH), lambda i: (i, 0))],
      out_specs=[
          pl.BlockSpec((BR, CH), lambda i: (i, 0)),
          pl.BlockSpec((BR, CH), lambda i: (i, 0)),
      ],
      out_shape=[
          jax.ShapeDtypeStruct((R, CH), jnp.int32),
          jax.ShapeDtypeStruct((R, CH), jnp.int32),
      ],
  )(idxf)


def _tc_preproc(gpre):
  nb = EP // BE

  def kfn(gd_ref, gs_ref, ea_ref, es_ref):
    gd = gd_ref[...]
    gs = gs_ref[...]
    rel = gs[:, 0:3] - gd[:, 0:3]
    n2 = jnp.sum(rel * rel, axis=1, keepdims=True)
    dist = jnp.sqrt(n2 + 1e-12)
    ea4 = _sh4(rel)
    pc = gs[:, 3:4] * gd[:, 3:4]
    one = jnp.ones((BE, 1), jnp.float32)
    zero = jnp.zeros((BE, 4), jnp.float32)
    ea_ref[...] = jnp.concatenate([ea4, one] + [zero] * 2 + [zero[:, :3]],
                                  axis=1)
    es_ref[...] = jnp.concatenate([ea4, ea4 * dist, ea4 * pc, zero], axis=1)

  return pl.pallas_call(
      kfn,
      grid=(nb,),
      in_specs=[
          pl.BlockSpec((BE, 16), lambda i: (i, 0)),
          pl.BlockSpec((BE, 16), lambda i: (i + nb, 0)),
      ],
      out_specs=[
          pl.BlockSpec((BE, 16), lambda i: (i, 0)),
          pl.BlockSpec((BE, 16), lambda i: (i, 0)),
      ],
      out_shape=[
          jax.ShapeDtypeStruct((EP, 16), jnp.float32),
          jax.ShapeDtypeStruct((EP, 16), jnp.float32),
      ],
  )(gpre, gpre)


def _tc_embed(nf, mp, na, w, b):
  def kfn(nf_ref, mp_ref, na_ref, w_ref, b_ref, x0_ref, nat_ref):
    nf = nf_ref[...]
    pos = nf[:, 0:3]
    vel = nf[:, 5:8]
    v2 = jnp.sum(vel * vel, axis=1, keepdims=True)
    vel_abs = jnp.sqrt(v2 + 1e-12)
    vel_emb = _sh4(vel)
    na_v = na_ref[...]
    nattr = na_v[:, 0:4] / jnp.clip(na_v[:, 4:5], 1.0, None) + vel_emb
    mp_v = mp_ref[...]
    mean = mp_v[:, 0:3] / jnp.clip(mp_v[:, 4:5], 1.0, None)
    feat = jnp.concatenate(
        [pos - mean, vel, vel_abs, jnp.zeros((BN, 1), jnp.float32)], axis=1)
    acc = jnp.zeros((BN, H), jnp.float32) + b_ref[...]
    for a in range(4):
      acc = acc + nattr[:, a:a + 1] * jnp.dot(
          feat, w_ref[a], preferred_element_type=jnp.float32)
    x0_ref[...] = acc
    nat_ref[...] = nattr

  nb = NP // BN
  return pl.pallas_call(
      kfn,
      grid=(nb,),
      in_specs=[
          pl.BlockSpec((BN, 16), lambda i: (i, 0)),
          pl.BlockSpec((BN, 16), lambda i: (i, 0)),
          pl.BlockSpec((BN, 16), lambda i: (i, 0)),
          pl.BlockSpec((4, 8, H), lambda i: (0, 0, 0)),
          pl.BlockSpec((1, H), lambda i: (0, 0)),
      ],
      out_specs=[
          pl.BlockSpec((BN, H), lambda i: (i, 0)),
          pl.BlockSpec((BN, 4), lambda i: (i, 0)),
      ],
      out_shape=[
          jax.ShapeDtypeStruct((NP, H), jnp.float32),
          jax.ShapeDtypeStruct((NP, 4), jnp.float32),
      ],
  )(nf, mp, na, w, b)


def _tc_edge(g, escal, w1c, wdp, b1, w2c, b2):
  nb = EP // BE

  def kfn(xi_ref, xj_ref, es_ref, w1_ref, wdp_ref, b1_ref, w2_ref, b2_ref,
          out_ref):
    xij = jnp.concatenate([xi_ref[...], xj_ref[...]], axis=1)
    es = es_ref[...]
    t = jnp.dot(xij, w1_ref[...], preferred_element_type=jnp.float32)
    acc = b1_ref[...] + jnp.dot(es[:, 4:12], wdp_ref[...],
                                preferred_element_type=jnp.float32)
    for a in range(4):
      acc = acc + es[:, a:a + 1] * t[:, a * H:(a + 1) * H]
    m1 = _silu(acc)
    t2 = jnp.dot(m1, w2_ref[...], preferred_element_type=jnp.float32)
    acc2 = jnp.zeros((BE, H), jnp.float32) + b2_ref[...]
    for a in range(4):
      acc2 = acc2 + es[:, a:a + 1] * t2[:, a * H:(a + 1) * H]
    out_ref[...] = _silu(acc2)

  return pl.pallas_call(
      kfn,
      grid=(nb,),
      in_specs=[
          pl.BlockSpec((BE, H), lambda i: (i, 0)),
          pl.BlockSpec((BE, H), lambda i: (i + nb, 0)),
          pl.BlockSpec((BE, 16), lambda i: (i, 0)),
          pl.BlockSpec((2 * H, 4 * H), lambda i: (0, 0)),
          pl.BlockSpec((8, H), lambda i: (0, 0)),
          pl.BlockSpec((1, H), lambda i: (0, 0)),
          pl.BlockSpec((H, 4 * H), lambda i: (0, 0)),
          pl.BlockSpec((1, H), lambda i: (0, 0)),
      ],
      out_specs=pl.BlockSpec((BE, H), lambda i: (i, 0)),
      out_shape=jax.ShapeDtypeStruct((EP, H), jnp.float32),
  )(g, g, escal, w1c, wdp, b1, w2c, b2)


def _tc_node(x, agg, nat, wu1c, b1, wu2c, b2):
  def kfn(x_ref, agg_ref, nat_ref, wu1_ref, b1_ref, wu2_ref,
          b2_ref, out_ref):
    x_v = x_ref[...]
    nat = nat_ref[...]
    xcat = jnp.concatenate([x_v, agg_ref[...]], axis=1)
    t = jnp.dot(xcat, wu1_ref[...], preferred_element_type=jnp.float32)
    acc = jnp.zeros((BN, H), jnp.float32) + b1_ref[...]
    for a in range(4):
      acc = acc + nat[:, a:a + 1] * t[:, a * H:(a + 1) * H]
    u = _silu(acc)
    t2 = jnp.dot(u, wu2_ref[...], preferred_element_type=jnp.float32)
    acc2 = jnp.zeros((BN, H), jnp.float32) + b2_ref[...]
    for a in range(4):
      acc2 = acc2 + nat[:, a:a + 1] * t2[:, a * H:(a + 1) * H]
    out_ref[...] = x_v + acc2

  nb = NP // BN
  return pl.pallas_call(
      kfn,
      grid=(nb,),
      in_specs=[
          pl.BlockSpec((BN, H), lambda i: (i, 0)),
          pl.BlockSpec((BN, H), lambda i: (i, 0)),
          pl.BlockSpec((BN, 4), lambda i: (i, 0)),
          pl.BlockSpec((2 * H, 4 * H), lambda i: (0, 0)),
          pl.BlockSpec((1, H), lambda i: (0, 0)),
          pl.BlockSpec((H, 4 * H), lambda i: (0, 0)),
          pl.BlockSpec((1, H), lambda i: (0, 0)),
      ],
      out_specs=pl.BlockSpec((BN, H), lambda i: (i, 0)),
      out_shape=jax.ShapeDtypeStruct((NP, H), jnp.float32),
  )(x, agg, nat, wu1c, b1, wu2c, b2)


def _tc_output(x, nat, nf, wo1, bo1, wo2, bo2):
  def kfn(x_ref, nat_ref, nf_ref, wo1_ref, bo1_ref, wo2_ref, bo2_ref,
          out_ref):
    x_v = x_ref[...]
    nat = nat_ref[...]
    acc = jnp.zeros((BN, H), jnp.float32) + bo1_ref[...]
    for a in range(4):
      acc = acc + nat[:, a:a + 1] * jnp.dot(
          x_v, wo1_ref[a], preferred_element_type=jnp.float32)
    u = _silu(acc)
    acc2 = jnp.zeros((BN, 128), jnp.float32) + bo2_ref[...]
    for a in range(4):
      acc2 = acc2 + nat[:, a:a + 1] * jnp.dot(
          u, wo2_ref[a], preferred_element_type=jnp.float32)
    pos = nf_ref[...][:, 0:3]
    out_ref[...] = acc2 + jnp.concatenate(
        [pos, jnp.zeros((BN, 125), jnp.float32)], axis=1)

  nb = NP // BN
  return pl.pallas_call(
      kfn,
      grid=(nb,),
      in_specs=[
          pl.BlockSpec((BN, H), lambda i: (i, 0)),
          pl.BlockSpec((BN, 4), lambda i: (i, 0)),
          pl.BlockSpec((BN, 16), lambda i: (i, 0)),
          pl.BlockSpec((4, H, H), lambda i: (0, 0, 0)),
          pl.BlockSpec((1, H), lambda i: (0, 0)),
          pl.BlockSpec((4, H, 128), lambda i: (0, 0, 0)),
          pl.BlockSpec((1, 128), lambda i: (0, 0)),
      ],
      out_specs=pl.BlockSpec((BN, 128), lambda i: (i, 0)),
      out_shape=jax.ShapeDtypeStruct((NP, 128), jnp.float32),
  )(x, nat, nf, wo1, bo1, wo2, bo2)


def _tp_weights(p):
  return p['W'].transpose(1, 0, 2), p['b'][None, :]


@jax.jit
def _run(pos, vel, charges, params, edge_index, batch):
  i32 = jnp.int32
  src = edge_index[0].astype(i32)
  dst = edge_index[1].astype(i32)
  batch = batch.astype(i32)

  nf = _tc_nf(pos, vel, charges)

  eidx_p = jnp.pad(jnp.stack([dst, src]), ((0, 0), (0, EP - E)))
  gidx = eidx_p.reshape(NC, NS, K_G, CH)

  s_lo, s_hi = _tc_remap(eidx_p[0].reshape(EP // CH, CH), E, S_NODE)
  sidx = jnp.stack([s_lo, s_hi]).reshape(NC, NS, K_E, CH)

  b_lo, b_hi = _tc_remap(
      jnp.pad(batch, (0, NP - N)).reshape(NP // CH, CH), N, S_G)
  bidx = jnp.stack([b_lo, b_hi]).reshape(NC, NS, K_B, CH)

  K_M = -(-NP // (NC * NS * CH))
  NMP = NC * NS * K_M * CH
  midx = jnp.minimum(jax.lax.iota(i32, NMP) // 5, G - 1)
  midx4 = midx.reshape(NC, NS, K_M, CH)

  z16 = jnp.zeros((CH, 16), jnp.float32)
  z64 = jnp.zeros((CH, 64), jnp.float32)

  gpre = _sc_gather(nf, gidx, 16, K_G)
  ea16, escal = _tc_preproc(gpre)
  na = _sc_scatter_add(ea16, sidx, z16, 16, K_E, S_NODE, ACC_N)
  mg = _sc_scatter_add(nf, bidx, z16, 16, K_B, S_G, ACC_G)
  mp = _sc_gather(mg, midx4, 16, K_M)[:NP]

  w_emb, b_emb = _tp_weights(params['emb'])
  w_emb = jnp.pad(w_emb, ((0, 0), (0, 1), (0, 0)))
  x, nat = _tc_embed(nf, mp, na, w_emb, b_emb)

  def _cat_w(w4):
    return w4.transpose(1, 0, 2).reshape(w4.shape[1], 4 * H)

  for lp in params['layers']:
    w1, b1 = _tp_weights(lp['m1'])
    w1c = _cat_w(jnp.concatenate([w1[:, :H], w1[:, H:2 * H]], axis=1))
    wdp = jnp.concatenate([w1[:, 2 * H], w1[:, 2 * H + 1]], axis=0)
    w2, b2 = _tp_weights(lp['m2'])
    wu1, bu1 = _tp_weights(lp['u1'])
    wu2, bu2 = _tp_weights(lp['u2'])

    g = _sc_gather(x, gidx, H, K_G)
    m2 = _tc_edge(g, escal, w1c, wdp, b1, _cat_w(w2), b2)
    agg = _sc_scatter_add(m2, sidx, z64, H, K_E, S_NODE, ACC_N)
    x = _tc_node(x, agg, nat, _cat_w(wu1), bu1, _cat_w(wu2), bu2)

  wo1, bo1 = _tp_weights(params['o1'])
  wo2, bo2 = _tp_weights(params['o2'])
  wo2 = jnp.pad(wo2, ((0, 0), (0, 0), (0, 125)))
  bo2 = jnp.pad(bo2, ((0, 0), (0, 125)))
  out = _tc_output(x, nat, nf, wo1, bo1, wo2, bo2)
  return out[:N, :3]


def kernel(pos, vel, charges, params, edge_index, batch):
  return _run(pos, vel, charges, params, edge_index, batch)


# speedup vs baseline: 1.6105x; 1.4220x over previous
import functools

import jax
import jax.numpy as jnp
from jax import lax
from jax.experimental import pallas as pl
from jax.experimental.pallas import tpu as pltpu
from jax.experimental.pallas import tpu_sc as plsc

N = 50000
E = 200000
G = 10000
H = 64

NC, NS = 2, 16
CH = 128

S_NODE = 25600
NP = 2 * S_NODE
ACC_N = 25728
EP = 200704
K_E = EP // NS // CH
K_G = 2 * EP // (NC * NS) // CH
S_G = 5120
GP = 2 * S_G
ACC_G = 5248
K_B = NP // NS // CH

BE = 1024
BN = 1024

@functools.cache
def _sc_mesh():
  return plsc.VectorSubcoreMesh(
      core_axis_name="c", subcore_axis_name="s", num_cores=NC, num_subcores=NS)


def _sc_gather(table, idx4, d, k, dtype=jnp.float32):
  rows_pt = k * CH

  def body(table_h, idx_h, out_h, idx_v, buf_a, buf_b, sem_a, sem_b):
    cid = lax.axis_index("c")
    sid = lax.axis_index("s")
    base = (cid * NS + sid) * rows_pt
    pltpu.sync_copy(idx_h.at[cid, sid], idx_v)

    def step(j, carry):
      @pl.when(lax.rem(j, 2) == 0)
      def _():
        pltpu.async_copy(table_h.at[idx_v.at[j]], buf_a, sem_a).wait()
        pltpu.sync_copy(buf_a, out_h.at[pl.ds(base + j * CH, CH)])

      @pl.when(lax.rem(j, 2) == 1)
      def _():
        pltpu.async_copy(table_h.at[idx_v.at[j]], buf_b, sem_b).wait()
        pltpu.sync_copy(buf_b, out_h.at[pl.ds(base + j * CH, CH)])
      return carry

    lax.fori_loop(0, k, step, 0, unroll=False)

  f = pl.kernel(
      body,
      out_type=jax.ShapeDtypeStruct((NC * NS * rows_pt, d), dtype),
      mesh=_sc_mesh(),
      compiler_params=pltpu.CompilerParams(use_tc_tiling_on_sc=False),
      scratch_types=[
          pltpu.VMEM((k, CH), jnp.int32),
          pltpu.VMEM((CH, d), dtype),
          pltpu.VMEM((CH, d), dtype),
          pltpu.SemaphoreType.DMA,
          pltpu.SemaphoreType.DMA,
      ],
  )
  return f(table, idx4)


def _sc_scatter_add(vals, idx4, zeros, d, k, s_half, acc_rows):
  z_pt = acc_rows // NS
  o_pt = s_half // NS
  nfull, rem = divmod(z_pt, CH)

  def body(vals_h, idx_h, zeros_h, out_h, idx_v, vbuf, acc):
    cid = lax.axis_index("c")
    sid = lax.axis_index("s")
    pltpu.sync_copy(idx_h.at[cid, sid], idx_v)

    zb = sid * z_pt
    for t in range(nfull):
      pltpu.sync_copy(zeros_h, acc.at[pl.ds(zb + t * CH, CH)])
    if rem:
      pltpu.sync_copy(zeros_h.at[pl.ds(0, rem)],
                      acc.at[pl.ds(zb + nfull * CH, rem)])
    plsc.subcore_barrier()

    tb = sid * (k * CH)

    def step(j, carry):
      pltpu.sync_copy(vals_h.at[pl.ds(tb + j * CH, CH)], vbuf)
      pltpu.sync_copy(vbuf, acc.at[idx_v.at[j]], add=True)
      return carry

    lax.fori_loop(0, k, step, 0, unroll=False)
    plsc.subcore_barrier()

    ob = sid * o_pt
    pltpu.sync_copy(acc.at[pl.ds(ob, o_pt)],
                    out_h.at[pl.ds(cid * s_half + ob, o_pt)])

  f = pl.kernel(
      body,
      out_type=jax.ShapeDtypeStruct((2 * s_half, d), jnp.float32),
      mesh=_sc_mesh(),
      compiler_params=pltpu.CompilerParams(use_tc_tiling_on_sc=False),
      scratch_types=[
          pltpu.VMEM((k, CH), jnp.int32),
          pltpu.VMEM((CH, d), jnp.float32),
          pltpu.VMEM_SHARED((acc_rows, d), jnp.float32),
      ],
  )
  return f(vals, idx4, zeros)


def _silu(x):
  return x * jax.nn.sigmoid(x)


def _sh4(r):
  n2 = jnp.sum(r * r, axis=1, keepdims=True)
  unit = r / jnp.clip(jnp.sqrt(n2), 1e-8, None)
  y0 = jnp.full((r.shape[0], 1), 0.28209479177387814, dtype=r.dtype)
  return jnp.concatenate([y0, 0.4886025119029199 * unit], axis=1)


def _tc_nf(pos, vel, charges):
  BR = 2000

  def kfn(p_ref, v_ref, c_ref, out_ref):
    one = jnp.ones((BR, 1), jnp.float32)
    zero = jnp.zeros((BR, 8), jnp.float32)
    out_ref[...] = jnp.concatenate(
        [p_ref[...], c_ref[...], one, v_ref[...], zero], axis=1)

  return pl.pallas_call(
      kfn,
      grid=(N // BR,),
      in_specs=[
          pl.BlockSpec((BR, 3), lambda i: (i, 0)),
          pl.BlockSpec((BR, 3), lambda i: (i, 0)),
          pl.BlockSpec((BR, 1), lambda i: (i, 0)),
      ],
      out_specs=pl.BlockSpec((BR, 16), lambda i: (i, 0)),
      out_shape=jax.ShapeDtypeStruct((NP, 16), jnp.float32),
  )(pos, vel, charges)


def _tc_remap(idxf, n_valid, s_half):
  R = idxf.shape[0]
  nb = 4 if R % 32 == 0 else 1
  BR = R // nb

  def kfn(i_ref, lo_ref, hi_ref):
    i = pl.program_id(0)
    v = i_ref[...]
    row = jax.lax.broadcasted_iota(jnp.int32, (BR, CH), 0) + i * BR
    col = jax.lax.broadcasted_iota(jnp.int32, (BR, CH), 1)
    valid = row * CH + col < n_valid
    lo_ref[...] = jnp.where(valid & (v < s_half), v, s_half)
    hi_ref[...] = jnp.where(valid & (v >= s_half), v - s_half, s_half)

  return pl.pallas_call(
      kfn,
      grid=(nb,),
      in_specs=[pl.BlockSpec((BR, CH), lambda i: (i, 0))],
      out_specs=[
          pl.BlockSpec((BR, CH), lambda i: (i, 0)),
          pl.BlockSpec((BR, CH), lambda i: (i, 0)),
      ],
      out_shape=[
          jax.ShapeDtypeStruct((R, CH), jnp.int32),
          jax.ShapeDtypeStruct((R, CH), jnp.int32),
      ],
  )(idxf)


def _tc_preproc(gpre):
  nb = EP // BE
  BH = BE // 2

  def kfn(gd_ref, gs_ref, ea_ref, es_ref):
    gd = gd_ref[0]
    gs = gs_ref[0]
    one = jnp.ones((BH, 1), jnp.float32)
    zero = jnp.zeros((BH, 4), jnp.float32)
    ea_h, es_h = [], []
    for o in (0, 16):
      rel = gs[:, o:o + 3] - gd[:, o:o + 3]
      n2 = jnp.sum(rel * rel, axis=1, keepdims=True)
      dist = jnp.sqrt(n2 + 1e-12)
      ea4 = _sh4(rel)
      pc = gs[:, o + 3:o + 4] * gd[:, o + 3:o + 4]
      ea_h += [ea4, one, zero, zero, zero[:, :3]]
      es_h += [ea4, ea4 * dist, ea4 * pc, zero]
    ea_ref[...] = jnp.concatenate(ea_h, axis=1)
    es_ref[...] = jnp.concatenate(es_h, axis=1)

  return pl.pallas_call(
      kfn,
      grid=(nb,),
      in_specs=[
          pl.BlockSpec((1, BH, 32), lambda i: (0, i, 0)),
          pl.BlockSpec((1, BH, 32), lambda i: (1, i, 0)),
      ],
      out_specs=[
          pl.BlockSpec((BH, 32), lambda i: (i, 0)),
          pl.BlockSpec((BH, 32), lambda i: (i, 0)),
      ],
      out_shape=[
          jax.ShapeDtypeStruct((EP // 2, 32), jnp.float32),
          jax.ShapeDtypeStruct((EP // 2, 32), jnp.float32),
      ],
  )(gpre, gpre)


def _tc_embed(nf, mp, na, w, b):
  BH = BN // 2

  def kfn(nf_ref, mp_ref, na_ref, w_ref, b_ref, x0_ref, nat_ref):
    nf = nf_ref[...]
    mp_v = mp_ref[...]
    na_v = na_ref[...]
    feat_h, nat_h = [], []
    for o in (0, 16):
      pos = nf[:, o:o + 3]
      vel = nf[:, o + 5:o + 8]
      v2 = jnp.sum(vel * vel, axis=1, keepdims=True)
      vel_abs = jnp.sqrt(v2 + 1e-12)
      vel_emb = _sh4(vel)
      nattr = (na_v[:, o:o + 4] / jnp.clip(na_v[:, o + 4:o + 5], 1.0, None)
               + vel_emb)
      mean = mp_v[:, o:o + 3] / jnp.clip(mp_v[:, o + 4:o + 5], 1.0, None)
      feat_h += [pos - mean, vel, vel_abs, jnp.zeros((BH, 1), jnp.float32)]
      nat_h.append(nattr)
    feat = jnp.concatenate(feat_h, axis=1)
    nat = jnp.concatenate(nat_h, axis=1)
    t = jnp.dot(feat, w_ref[...], preferred_element_type=jnp.float32)
    acc = jnp.zeros((BH, 128), jnp.float32) + b_ref[...]
    for a in range(4):
      natw = jnp.concatenate(
          [jnp.broadcast_to(nat[:, a:a + 1], (BH, H)),
           jnp.broadcast_to(nat[:, 4 + a:5 + a], (BH, H))], axis=1)
      acc = acc + natw * t[:, a * 128:(a + 1) * 128]
    x0_ref[...] = acc
    nat_ref[...] = nat

  nb = NP // BN
  return pl.pallas_call(
      kfn,
      grid=(nb,),
      in_specs=[
          pl.BlockSpec((BH, 32), lambda i: (i, 0)),
          pl.BlockSpec((BH, 32), lambda i: (i, 0)),
          pl.BlockSpec((BH, 32), lambda i: (i, 0)),
          pl.BlockSpec((16, 512), lambda i: (0, 0)),
          pl.BlockSpec((1, 128), lambda i: (0, 0)),
      ],
      out_specs=[
          pl.BlockSpec((BH, 128), lambda i: (i, 0)),
          pl.BlockSpec((BH, 8), lambda i: (i, 0)),
      ],
      out_shape=[
          jax.ShapeDtypeStruct((NP // 2, 128), jnp.float32),
          jax.ShapeDtypeStruct((NP // 2, 8), jnp.float32),
      ],
  )(nf, mp, na, w, b)


def _tc_edge(g, escal, w1c, wsc, wdp, b1, w2c, b2):
  nb = EP // BE

  BH = BE // 2

  def kfn(xi_ref, xj_ref, es_ref, wd_ref, ws_ref, wdp_ref, b1_ref, w2_ref,
          b2_ref, out_ref):
    xd = xi_ref[0]
    xs = xj_ref[0]
    es = es_ref[...]
    t = (jnp.dot(xd, wd_ref[...], preferred_element_type=jnp.float32)
         + jnp.dot(xs, ws_ref[...], preferred_element_type=jnp.float32))
    esdp = jnp.concatenate([es[:, 4:12], es[:, 20:28]], axis=1)
    acc = b1_ref[...] + jnp.dot(esdp, wdp_ref[...],
                                preferred_element_type=jnp.float32)
    esw = []
    for a in range(4):
      esw.append(jnp.concatenate(
          [jnp.broadcast_to(es[:, a:a + 1], (BH, H)),
           jnp.broadcast_to(es[:, 16 + a:17 + a], (BH, H))], axis=1))
      acc = acc + esw[a] * t[:, a * 128:(a + 1) * 128]
    m1 = _silu(acc)
    t2 = jnp.dot(m1, w2_ref[...], preferred_element_type=jnp.float32)
    acc2 = jnp.zeros((BH, 128), jnp.float32) + b2_ref[...]
    for a in range(4):
      acc2 = acc2 + esw[a] * t2[:, a * 128:(a + 1) * 128]
    out_ref[...] = _silu(acc2)

  return pl.pallas_call(
      kfn,
      grid=(nb,),
      in_specs=[
          pl.BlockSpec((1, BH, 128), lambda i: (0, i, 0)),
          pl.BlockSpec((1, BH, 128), lambda i: (1, i, 0)),
          pl.BlockSpec((BH, 32), lambda i: (i, 0)),
          pl.BlockSpec((128, 512), lambda i: (0, 0)),
          pl.BlockSpec((128, 512), lambda i: (0, 0)),
          pl.BlockSpec((16, 128), lambda i: (0, 0)),
          pl.BlockSpec((1, 128), lambda i: (0, 0)),
          pl.BlockSpec((128, 512), lambda i: (0, 0)),
          pl.BlockSpec((1, 128), lambda i: (0, 0)),
      ],
      out_specs=pl.BlockSpec((BH, 128), lambda i: (i, 0)),
      out_shape=jax.ShapeDtypeStruct((EP // 2, 128), jnp.float32),
  )(g, g, escal, w1c, wsc, wdp, b1, w2c, b2)


def _tc_node(x, agg, nat, wuxc, wuac, b1, wu2c, b2):
  BH = BN // 2

  def kfn(x_ref, agg_ref, nat_ref, wux_ref, wua_ref, b1_ref, wu2_ref,
          b2_ref, out_ref):
    x_v = x_ref[...]
    nat = nat_ref[...]
    t = (jnp.dot(x_v, wux_ref[...], preferred_element_type=jnp.float32)
         + jnp.dot(agg_ref[...], wua_ref[...],
                   preferred_element_type=jnp.float32))
    acc = jnp.zeros((BH, 128), jnp.float32) + b1_ref[...]
    natw = []
    for a in range(4):
      natw.append(jnp.concatenate(
          [jnp.broadcast_to(nat[:, a:a + 1], (BH, H)),
           jnp.broadcast_to(nat[:, 4 + a:5 + a], (BH, H))], axis=1))
      acc = acc + natw[a] * t[:, a * 128:(a + 1) * 128]
    u = _silu(acc)
    t2 = jnp.dot(u, wu2_ref[...], preferred_element_type=jnp.float32)
    acc2 = jnp.zeros((BH, 128), jnp.float32) + b2_ref[...]
    for a in range(4):
      acc2 = acc2 + natw[a] * t2[:, a * 128:(a + 1) * 128]
    out_ref[...] = x_v + acc2

  nb = NP // BN
  return pl.pallas_call(
      kfn,
      grid=(nb,),
      in_specs=[
          pl.BlockSpec((BH, 128), lambda i: (i, 0)),
          pl.BlockSpec((BH, 128), lambda i: (i, 0)),
          pl.BlockSpec((BH, 8), lambda i: (i, 0)),
          pl.BlockSpec((128, 512), lambda i: (0, 0)),
          pl.BlockSpec((128, 512), lambda i: (0, 0)),
          pl.BlockSpec((1, 128), lambda i: (0, 0)),
          pl.BlockSpec((128, 512), lambda i: (0, 0)),
          pl.BlockSpec((1, 128), lambda i: (0, 0)),
      ],
      out_specs=pl.BlockSpec((BH, 128), lambda i: (i, 0)),
      out_shape=jax.ShapeDtypeStruct((NP // 2, 128), jnp.float32),
  )(x, agg, nat, wuxc, wuac, b1, wu2c, b2)


def _tc_output(x, nat, nf, wo1, bo1, wo2, bo2):
  BH = BN // 2

  def kfn(x_ref, nat_ref, nf_ref, wo1_ref, bo1_ref, wo2_ref, bo2_ref,
          out_ref):
    x_v = x_ref[...]
    nat = nat_ref[...]
    nf = nf_ref[...]
    t = jnp.dot(x_v, wo1_ref[...], preferred_element_type=jnp.float32)
    acc = jnp.zeros((BH, 128), jnp.float32) + bo1_ref[...]
    natw = []
    for a in range(4):
      natw.append(jnp.concatenate(
          [jnp.broadcast_to(nat[:, a:a + 1], (BH, H)),
           jnp.broadcast_to(nat[:, 4 + a:5 + a], (BH, H))], axis=1))
      acc = acc + natw[a] * t[:, a * 128:(a + 1) * 128]
    u = _silu(acc)
    t2 = jnp.dot(u, wo2_ref[...], preferred_element_type=jnp.float32)
    acc2 = jnp.zeros((BH, 256), jnp.float32) + bo2_ref[...]
    for a in range(4):
      nw = jnp.concatenate(
          [jnp.broadcast_to(nat[:, a:a + 1], (BH, 128)),
           jnp.broadcast_to(nat[:, 4 + a:5 + a], (BH, 128))], axis=1)
      acc2 = acc2 + nw * t2[:, a * 256:(a + 1) * 256]
    z125 = jnp.zeros((BH, 125), jnp.float32)
    out_ref[...] = acc2 + jnp.concatenate(
        [nf[:, 0:3], z125, nf[:, 16:19], z125], axis=1)

  nb = NP // BN
  return pl.pallas_call(
      kfn,
      grid=(nb,),
      in_specs=[
          pl.BlockSpec((BH, 128), lambda i: (i, 0)),
          pl.BlockSpec((BH, 8), lambda i: (i, 0)),
          pl.BlockSpec((BH, 32), lambda i: (i, 0)),
          pl.BlockSpec((128, 512), lambda i: (0, 0)),
          pl.BlockSpec((1, 128), lambda i: (0, 0)),
          pl.BlockSpec((128, 1024), lambda i: (0, 0)),
          pl.BlockSpec((1, 256), lambda i: (0, 0)),
      ],
      out_specs=pl.BlockSpec((BH, 256), lambda i: (i, 0)),
      out_shape=jax.ShapeDtypeStruct((NP // 2, 256), jnp.float32),
  )(x, nat, nf, wo1, bo1, wo2, bo2)


def _tp_weights(p):
  return p['W'].transpose(1, 0, 2), p['b'][None, :]


def _bd(w4):
  z = jnp.zeros_like(w4)
  top = jnp.concatenate([w4, z], axis=2)
  bot = jnp.concatenate([z, w4], axis=2)
  bd = jnp.concatenate([top, bot], axis=1)
  return bd.transpose(1, 0, 2).reshape(bd.shape[1], -1)


def _bd1(w):
  z = jnp.zeros_like(w)
  return jnp.concatenate(
      [jnp.concatenate([w, z], 1), jnp.concatenate([z, w], 1)], 0)


def _b2(b):
  return jnp.concatenate([b, b], axis=1)


@jax.jit
def _run(pos, vel, charges, params, edge_index, batch):
  i32 = jnp.int32
  src = edge_index[0].astype(i32)
  dst = edge_index[1].astype(i32)
  batch = batch.astype(i32)

  nf = _tc_nf(pos, vel, charges)

  eidx_p = jnp.pad(jnp.stack([dst, src]), ((0, 0), (0, EP - E)))
  gidx = eidx_p.reshape(NC, NS, K_G, CH)

  s_lo, s_hi = _tc_remap(eidx_p[0].reshape(EP // CH, CH), E, S_NODE)
  sidx = jnp.stack([s_lo, s_hi]).reshape(NC, NS, K_E, CH)

  b_lo, b_hi = _tc_remap(
      jnp.pad(batch, (0, NP - N)).reshape(NP // CH, CH), N, S_G)
  bidx = jnp.stack([b_lo, b_hi]).reshape(NC, NS, K_B, CH)

  K_M = -(-NP // (NC * NS * CH))
  NMP = NC * NS * K_M * CH
  midx = jnp.minimum(jax.lax.iota(i32, NMP) // 5, G - 1)
  midx4 = midx.reshape(NC, NS, K_M, CH)

  z16 = jnp.zeros((CH, 16), jnp.float32)
  z64 = jnp.zeros((CH, 64), jnp.float32)

  gpre = _sc_gather(nf, gidx, 16, K_G).reshape(2, EP // 2, 32)
  ea16p, escal = _tc_preproc(gpre)
  na = _sc_scatter_add(ea16p.reshape(EP, 16), sidx, z16, 16, K_E,
                       S_NODE, ACC_N)
  mg = _sc_scatter_add(nf, bidx, z16, 16, K_B, S_G, ACC_G)
  mp = _sc_gather(mg, midx4, 16, K_M)[:NP]

  w_emb, b_emb = _tp_weights(params['emb'])
  w_emb = jnp.pad(w_emb, ((0, 0), (0, 1), (0, 0)))
  x, nat = _tc_embed(nf.reshape(NP // 2, 32), mp.reshape(NP // 2, 32),
                     na.reshape(NP // 2, 32), _bd(w_emb), _b2(b_emb))

  for lp in params['layers']:
    w1, b1 = _tp_weights(lp['m1'])
    wdp = jnp.concatenate([w1[:, 2 * H], w1[:, 2 * H + 1]], axis=0)
    w2, b2 = _tp_weights(lp['m2'])
    wu1, bu1 = _tp_weights(lp['u1'])
    wu2, bu2 = _tp_weights(lp['u2'])

    g3 = _sc_gather(x.reshape(NP, H), gidx, H, K_G).reshape(2, EP // 2, 128)
    m2p = _tc_edge(g3, escal, _bd(w1[:, :H]), _bd(w1[:, H:2 * H]),
                   _bd1(wdp), _b2(b1), _bd(w2), _b2(b2))
    agg = _sc_scatter_add(m2p.reshape(EP, H), sidx, z64, H, K_E,
                          S_NODE, ACC_N).reshape(NP // 2, 128)
    x = _tc_node(x, agg, nat, _bd(wu1[:, :H]), _bd(wu1[:, H:]),
                 _b2(bu1), _bd(wu2), _b2(bu2))

  wo1, bo1 = _tp_weights(params['o1'])
  wo2, bo2 = _tp_weights(params['o2'])
  wo2 = jnp.pad(wo2, ((0, 0), (0, 0), (0, 125)))
  bo2 = jnp.pad(bo2, ((0, 0), (0, 125)))
  out = _tc_output(x, nat, nf.reshape(NP // 2, 32),
                   _bd(wo1), _b2(bo1), _bd(wo2), _b2(bo2))
  return out.reshape(NP, 128)[:N, :3]


def kernel(pos, vel, charges, params, edge_index, batch):
  return _run(pos, vel, charges, params, edge_index, batch)


# speedup vs baseline: 1.9227x; 1.1938x over previous
import functools

import jax
import jax.numpy as jnp
from jax import lax
from jax.experimental import pallas as pl
from jax.experimental.pallas import tpu as pltpu
from jax.experimental.pallas import tpu_sc as plsc

N = 50000
E = 200000
G = 10000
H = 64

NC, NS = 2, 16
CH = 128

S_NODE = 25600
NP = 2 * S_NODE
ACC_N = 25728
EP = 200704
K_E = EP // NS // CH
K_G = 2 * EP // (NC * NS) // CH
S_G = 5120
GP = 2 * S_G
ACC_G = 5248
K_B = NP // NS // CH

BE = 1024
BN = 1024

@functools.cache
def _sc_mesh():
  return plsc.VectorSubcoreMesh(
      core_axis_name="c", subcore_axis_name="s", num_cores=NC, num_subcores=NS)


def _sc_gather(table, idx4, d, k, dtype=jnp.float32):
  rows_pt = k * CH

  def body(table_h, idx_h, out_h, idx_v, buf_a, buf_b, sem_a, sem_b):
    cid = lax.axis_index("c")
    sid = lax.axis_index("s")
    base = (cid * NS + sid) * rows_pt
    pltpu.sync_copy(idx_h.at[cid, sid], idx_v)

    def step(j, carry):
      @pl.when(lax.rem(j, 2) == 0)
      def _():
        pltpu.async_copy(table_h.at[idx_v.at[j]], buf_a, sem_a).wait()
        pltpu.sync_copy(buf_a, out_h.at[pl.ds(base + j * CH, CH)])

      @pl.when(lax.rem(j, 2) == 1)
      def _():
        pltpu.async_copy(table_h.at[idx_v.at[j]], buf_b, sem_b).wait()
        pltpu.sync_copy(buf_b, out_h.at[pl.ds(base + j * CH, CH)])
      return carry

    lax.fori_loop(0, k, step, 0, unroll=False)

  f = pl.kernel(
      body,
      out_type=jax.ShapeDtypeStruct((NC * NS * rows_pt, d), dtype),
      mesh=_sc_mesh(),
      compiler_params=pltpu.CompilerParams(use_tc_tiling_on_sc=False),
      scratch_types=[
          pltpu.VMEM((k, CH), jnp.int32),
          pltpu.VMEM((CH, d), dtype),
          pltpu.VMEM((CH, d), dtype),
          pltpu.SemaphoreType.DMA,
          pltpu.SemaphoreType.DMA,
      ],
  )
  return f(table, idx4)


def _sc_scatter_add(vals, idx4, zeros, d, k, s_half, acc_rows):
  z_pt = acc_rows // NS
  o_pt = s_half // NS
  nfull, rem = divmod(z_pt, CH)

  def body(vals_h, idx_h, zeros_h, out_h, idx_v, vbuf, acc):
    cid = lax.axis_index("c")
    sid = lax.axis_index("s")
    pltpu.sync_copy(idx_h.at[cid, sid], idx_v)

    zb = sid * z_pt
    for t in range(nfull):
      pltpu.sync_copy(zeros_h, acc.at[pl.ds(zb + t * CH, CH)])
    if rem:
      pltpu.sync_copy(zeros_h.at[pl.ds(0, rem)],
                      acc.at[pl.ds(zb + nfull * CH, rem)])
    plsc.subcore_barrier()

    tb = sid * (k * CH)

    def step(j, carry):
      pltpu.sync_copy(vals_h.at[pl.ds(tb + j * CH, CH)], vbuf)
      pltpu.sync_copy(vbuf, acc.at[idx_v.at[j]], add=True)
      return carry

    lax.fori_loop(0, k, step, 0, unroll=False)
    plsc.subcore_barrier()

    ob = sid * o_pt
    pltpu.sync_copy(acc.at[pl.ds(ob, o_pt)],
                    out_h.at[pl.ds(cid * s_half + ob, o_pt)])

  f = pl.kernel(
      body,
      out_type=jax.ShapeDtypeStruct((2 * s_half, d), jnp.float32),
      mesh=_sc_mesh(),
      compiler_params=pltpu.CompilerParams(use_tc_tiling_on_sc=False),
      scratch_types=[
          pltpu.VMEM((k, CH), jnp.int32),
          pltpu.VMEM((CH, d), jnp.float32),
          pltpu.VMEM_SHARED((acc_rows, d), jnp.float32),
      ],
  )
  return f(vals, idx4, zeros)


def _silu(x):
  return x * jax.nn.sigmoid(x)


def _sh4(r):
  n2 = jnp.sum(r * r, axis=1, keepdims=True)
  unit = r / jnp.clip(jnp.sqrt(n2), 1e-8, None)
  y0 = jnp.full((r.shape[0], 1), 0.28209479177387814, dtype=r.dtype)
  return jnp.concatenate([y0, 0.4886025119029199 * unit], axis=1)


def _tc_nf(pos, vel, charges):
  BR = 2000

  def kfn(p_ref, v_ref, c_ref, out_ref):
    one = jnp.ones((BR, 1), jnp.float32)
    zero = jnp.zeros((BR, 8), jnp.float32)
    out_ref[...] = jnp.concatenate(
        [p_ref[...], c_ref[...], one, v_ref[...], zero], axis=1)

  return pl.pallas_call(
      kfn,
      grid=(N // BR,),
      in_specs=[
          pl.BlockSpec((BR, 3), lambda i: (i, 0)),
          pl.BlockSpec((BR, 3), lambda i: (i, 0)),
          pl.BlockSpec((BR, 1), lambda i: (i, 0)),
      ],
      out_specs=pl.BlockSpec((BR, 16), lambda i: (i, 0)),
      out_shape=jax.ShapeDtypeStruct((NP, 16), jnp.float32),
  )(pos, vel, charges)


def _tc_remap(idxf, n_valid, s_half):
  R = idxf.shape[0]
  nb = 4 if R % 32 == 0 else 1
  BR = R // nb

  def kfn(i_ref, lo_ref, hi_ref):
    i = pl.program_id(0)
    v = i_ref[...]
    row = jax.lax.broadcasted_iota(jnp.int32, (BR, CH), 0) + i * BR
    col = jax.lax.broadcasted_iota(jnp.int32, (BR, CH), 1)
    valid = row * CH + col < n_valid
    lo_ref[...] = jnp.where(valid & (v < s_half), v, s_half)
    hi_ref[...] = jnp.where(valid & (v >= s_half), v - s_half, s_half)

  return pl.pallas_call(
      kfn,
      grid=(nb,),
      in_specs=[pl.BlockSpec((BR, CH), lambda i: (i, 0))],
      out_specs=[
          pl.BlockSpec((BR, CH), lambda i: (i, 0)),
          pl.BlockSpec((BR, CH), lambda i: (i, 0)),
      ],
      out_shape=[
          jax.ShapeDtypeStruct((R, CH), jnp.int32),
          jax.ShapeDtypeStruct((R, CH), jnp.int32),
      ],
  )(idxf)


def _tc_preproc(gpre):
  nb = EP // BE
  BH = BE // 2

  def kfn(gd_ref, gs_ref, ea_ref, es_ref):
    gd = gd_ref[0]
    gs = gs_ref[0]
    one = jnp.ones((BH, 1), jnp.float32)
    zero = jnp.zeros((BH, 4), jnp.float32)
    ea_h, es_h = [], []
    for o in (0, 16):
      rel = gs[:, o:o + 3] - gd[:, o:o + 3]
      n2 = jnp.sum(rel * rel, axis=1, keepdims=True)
      dist = jnp.sqrt(n2 + 1e-12)
      ea4 = _sh4(rel)
      pc = gs[:, o + 3:o + 4] * gd[:, o + 3:o + 4]
      ea_h += [ea4, one, zero, zero, zero[:, :3]]
      es_h += [ea4, ea4 * dist, ea4 * pc, zero]
    ea_ref[...] = jnp.concatenate(ea_h, axis=1)
    es_ref[...] = jnp.concatenate(es_h, axis=1)

  return pl.pallas_call(
      kfn,
      grid=(nb,),
      in_specs=[
          pl.BlockSpec((1, BH, 32), lambda i: (0, i, 0)),
          pl.BlockSpec((1, BH, 32), lambda i: (1, i, 0)),
      ],
      out_specs=[
          pl.BlockSpec((BH, 32), lambda i: (i, 0)),
          pl.BlockSpec((BH, 32), lambda i: (i, 0)),
      ],
      out_shape=[
          jax.ShapeDtypeStruct((EP // 2, 32), jnp.float32),
          jax.ShapeDtypeStruct((EP // 2, 32), jnp.float32),
      ],
  )(gpre, gpre)


def _tc_embed(nf, mp, na, w, b):
  BH = BN // 2

  def kfn(nf_ref, mp_ref, na_ref, w_ref, b_ref, x0_ref, nat_ref):
    nf = nf_ref[...]
    mp_v = mp_ref[...]
    na_v = na_ref[...]
    feat_h, nat_h = [], []
    for o in (0, 16):
      pos = nf[:, o:o + 3]
      vel = nf[:, o + 5:o + 8]
      v2 = jnp.sum(vel * vel, axis=1, keepdims=True)
      vel_abs = jnp.sqrt(v2 + 1e-12)
      vel_emb = _sh4(vel)
      nattr = (na_v[:, o:o + 4] / jnp.clip(na_v[:, o + 4:o + 5], 1.0, None)
               + vel_emb)
      mean = mp_v[:, o:o + 3] / jnp.clip(mp_v[:, o + 4:o + 5], 1.0, None)
      feat_h += [pos - mean, vel, vel_abs, jnp.zeros((BH, 1), jnp.float32)]
      nat_h.append(nattr)
    feat = jnp.concatenate(feat_h, axis=1)
    nat = jnp.concatenate(nat_h, axis=1)
    t = jnp.dot(feat, w_ref[...], preferred_element_type=jnp.float32)
    acc = jnp.zeros((BH, 128), jnp.float32) + b_ref[...]
    for a in range(4):
      natw = jnp.concatenate(
          [jnp.broadcast_to(nat[:, a:a + 1], (BH, H)),
           jnp.broadcast_to(nat[:, 4 + a:5 + a], (BH, H))], axis=1)
      acc = acc + natw * t[:, a * 128:(a + 1) * 128]
    x0_ref[...] = acc
    nat_ref[...] = nat

  nb = NP // BN
  return pl.pallas_call(
      kfn,
      grid=(nb,),
      in_specs=[
          pl.BlockSpec((BH, 32), lambda i: (i, 0)),
          pl.BlockSpec((BH, 32), lambda i: (i, 0)),
          pl.BlockSpec((BH, 32), lambda i: (i, 0)),
          pl.BlockSpec((16, 512), lambda i: (0, 0)),
          pl.BlockSpec((1, 128), lambda i: (0, 0)),
      ],
      out_specs=[
          pl.BlockSpec((BH, 128), lambda i: (i, 0)),
          pl.BlockSpec((BH, 8), lambda i: (i, 0)),
      ],
      out_shape=[
          jax.ShapeDtypeStruct((NP // 2, 128), jnp.float32),
          jax.ShapeDtypeStruct((NP // 2, 8), jnp.float32),
      ],
  )(nf, mp, na, w, b)


def _tc_edge(g, escal, off, w1c, wsc, wdp, b1, w2c, b2):
  nb = EP // BE

  BH = BE // 2

  def kfn(xi_ref, xj_ref, es_ref, wd_ref, ws_ref, wdp_ref, b1_ref, w2_ref,
          b2_ref, out_ref):
    xd = xi_ref[0]
    xs = xj_ref[0]
    es = es_ref[...]
    t = (jnp.dot(xd, wd_ref[...], preferred_element_type=jnp.float32)
         + jnp.dot(xs, ws_ref[...], preferred_element_type=jnp.float32))
    esdp = jnp.concatenate([es[:, 4:12], es[:, 20:28]], axis=1)
    acc = b1_ref[...] + jnp.dot(esdp, wdp_ref[...],
                                preferred_element_type=jnp.float32)
    esw = []
    for a in range(4):
      esw.append(jnp.concatenate(
          [jnp.broadcast_to(es[:, a:a + 1], (BH, H)),
           jnp.broadcast_to(es[:, 16 + a:17 + a], (BH, H))], axis=1))
      acc = acc + esw[a] * t[:, a * 128:(a + 1) * 128]
    m1 = _silu(acc)
    t2 = jnp.dot(m1, w2_ref[...], preferred_element_type=jnp.float32)
    acc2 = jnp.zeros((BH, 128), jnp.float32) + b2_ref[...]
    for a in range(4):
      acc2 = acc2 + esw[a] * t2[:, a * 128:(a + 1) * 128]
    out_ref[...] = _silu(acc2)

  ne = g.shape[1]
  return pl.pallas_call(
      kfn,
      grid=(ne // BH,),
      in_specs=[
          pl.BlockSpec((1, BH, 128), lambda i: (0, i, 0)),
          pl.BlockSpec((1, BH, 128), lambda i: (1, i, 0)),
          pl.BlockSpec((BH, 32), lambda i: (i + off, 0)),
          pl.BlockSpec((128, 512), lambda i: (0, 0)),
          pl.BlockSpec((128, 512), lambda i: (0, 0)),
          pl.BlockSpec((16, 128), lambda i: (0, 0)),
          pl.BlockSpec((1, 128), lambda i: (0, 0)),
          pl.BlockSpec((128, 512), lambda i: (0, 0)),
          pl.BlockSpec((1, 128), lambda i: (0, 0)),
      ],
      out_specs=pl.BlockSpec((BH, 128), lambda i: (i, 0)),
      out_shape=jax.ShapeDtypeStruct((ne, 128), jnp.float32),
  )(g, g, escal, w1c, wsc, wdp, b1, w2c, b2)


def _tc_node(x, agg_a, agg_b, nat, wuxc, wuac, b1, wu2c, b2):
  BH = BN // 2

  def kfn(x_ref, aga_ref, agb_ref, nat_ref, wux_ref, wua_ref, b1_ref,
          wu2_ref, b2_ref, out_ref):
    x_v = x_ref[...]
    nat = nat_ref[...]
    agg = aga_ref[...] + agb_ref[...]
    t = (jnp.dot(x_v, wux_ref[...], preferred_element_type=jnp.float32)
         + jnp.dot(agg, wua_ref[...],
                   preferred_element_type=jnp.float32))
    acc = jnp.zeros((BH, 128), jnp.float32) + b1_ref[...]
    natw = []
    for a in range(4):
      natw.append(jnp.concatenate(
          [jnp.broadcast_to(nat[:, a:a + 1], (BH, H)),
           jnp.broadcast_to(nat[:, 4 + a:5 + a], (BH, H))], axis=1))
      acc = acc + natw[a] * t[:, a * 128:(a + 1) * 128]
    u = _silu(acc)
    t2 = jnp.dot(u, wu2_ref[...], preferred_element_type=jnp.float32)
    acc2 = jnp.zeros((BH, 128), jnp.float32) + b2_ref[...]
    for a in range(4):
      acc2 = acc2 + natw[a] * t2[:, a * 128:(a + 1) * 128]
    out_ref[...] = x_v + acc2

  nb = NP // BN
  return pl.pallas_call(
      kfn,
      grid=(nb,),
      in_specs=[
          pl.BlockSpec((BH, 128), lambda i: (i, 0)),
          pl.BlockSpec((BH, 128), lambda i: (i, 0)),
          pl.BlockSpec((BH, 128), lambda i: (i, 0)),
          pl.BlockSpec((BH, 8), lambda i: (i, 0)),
          pl.BlockSpec((128, 512), lambda i: (0, 0)),
          pl.BlockSpec((128, 512), lambda i: (0, 0)),
          pl.BlockSpec((1, 128), lambda i: (0, 0)),
          pl.BlockSpec((128, 512), lambda i: (0, 0)),
          pl.BlockSpec((1, 128), lambda i: (0, 0)),
      ],
      out_specs=pl.BlockSpec((BH, 128), lambda i: (i, 0)),
      out_shape=jax.ShapeDtypeStruct((NP // 2, 128), jnp.float32),
  )(x, agg_a, agg_b, nat, wuxc, wuac, b1, wu2c, b2)


def _tc_output(x, nat, nf, wo1, bo1, wo2, bo2):
  BH = BN // 2

  def kfn(x_ref, nat_ref, nf_ref, wo1_ref, bo1_ref, wo2_ref, bo2_ref,
          out_ref):
    x_v = x_ref[...]
    nat = nat_ref[...]
    nf = nf_ref[...]
    t = jnp.dot(x_v, wo1_ref[...], preferred_element_type=jnp.float32)
    acc = jnp.zeros((BH, 128), jnp.float32) + bo1_ref[...]
    natw = []
    for a in range(4):
      natw.append(jnp.concatenate(
          [jnp.broadcast_to(nat[:, a:a + 1], (BH, H)),
           jnp.broadcast_to(nat[:, 4 + a:5 + a], (BH, H))], axis=1))
      acc = acc + natw[a] * t[:, a * 128:(a + 1) * 128]
    u = _silu(acc)
    t2 = jnp.dot(u, wo2_ref[...], preferred_element_type=jnp.float32)
    acc2 = jnp.zeros((BH, 256), jnp.float32) + bo2_ref[...]
    for a in range(4):
      nw = jnp.concatenate(
          [jnp.broadcast_to(nat[:, a:a + 1], (BH, 128)),
           jnp.broadcast_to(nat[:, 4 + a:5 + a], (BH, 128))], axis=1)
      acc2 = acc2 + nw * t2[:, a * 256:(a + 1) * 256]
    z125 = jnp.zeros((BH, 125), jnp.float32)
    out_ref[...] = acc2 + jnp.concatenate(
        [nf[:, 0:3], z125, nf[:, 16:19], z125], axis=1)

  nb = NP // BN
  return pl.pallas_call(
      kfn,
      grid=(nb,),
      in_specs=[
          pl.BlockSpec((BH, 128), lambda i: (i, 0)),
          pl.BlockSpec((BH, 8), lambda i: (i, 0)),
          pl.BlockSpec((BH, 32), lambda i: (i, 0)),
          pl.BlockSpec((128, 512), lambda i: (0, 0)),
          pl.BlockSpec((1, 128), lambda i: (0, 0)),
          pl.BlockSpec((128, 1024), lambda i: (0, 0)),
          pl.BlockSpec((1, 256), lambda i: (0, 0)),
      ],
      out_specs=pl.BlockSpec((BH, 256), lambda i: (i, 0)),
      out_shape=jax.ShapeDtypeStruct((NP // 2, 256), jnp.float32),
  )(x, nat, nf, wo1, bo1, wo2, bo2)


def _tp_weights(p):
  return p['W'].transpose(1, 0, 2), p['b'][None, :]


def _bd(w4):
  z = jnp.zeros_like(w4)
  top = jnp.concatenate([w4, z], axis=2)
  bot = jnp.concatenate([z, w4], axis=2)
  bd = jnp.concatenate([top, bot], axis=1)
  return bd.transpose(1, 0, 2).reshape(bd.shape[1], -1)


def _bd1(w):
  z = jnp.zeros_like(w)
  return jnp.concatenate(
      [jnp.concatenate([w, z], 1), jnp.concatenate([z, w], 1)], 0)


def _b2(b):
  return jnp.concatenate([b, b], axis=1)


@jax.jit
def _run(pos, vel, charges, params, edge_index, batch):
  i32 = jnp.int32
  src = edge_index[0].astype(i32)
  dst = edge_index[1].astype(i32)
  batch = batch.astype(i32)

  nf = _tc_nf(pos, vel, charges)

  eidx_p = jnp.pad(jnp.stack([dst, src]), ((0, 0), (0, EP - E)))
  gidx = eidx_p.reshape(NC, NS, K_G, CH)
  EH = EP // 2
  gidx_a = eidx_p[:, :EH].reshape(NC, NS, K_G // 2, CH)
  gidx_b = eidx_p[:, EH:].reshape(NC, NS, K_G // 2, CH)

  s_lo, s_hi = _tc_remap(eidx_p[0].reshape(EP // CH, CH), E, S_NODE)
  sidx = jnp.stack([s_lo, s_hi]).reshape(NC, NS, K_E, CH)
  RH = EH // CH
  sidx_a = jnp.stack([s_lo[:RH], s_hi[:RH]]).reshape(NC, NS, K_E // 2, CH)
  sidx_b = jnp.stack([s_lo[RH:], s_hi[RH:]]).reshape(NC, NS, K_E // 2, CH)

  b_lo, b_hi = _tc_remap(
      jnp.pad(batch, (0, NP - N)).reshape(NP // CH, CH), N, S_G)
  bidx = jnp.stack([b_lo, b_hi]).reshape(NC, NS, K_B, CH)

  K_M = -(-NP // (NC * NS * CH))
  NMP = NC * NS * K_M * CH
  midx = jnp.minimum(jax.lax.iota(i32, NMP) // 5, G - 1)
  midx4 = midx.reshape(NC, NS, K_M, CH)

  z16 = jnp.zeros((CH, 16), jnp.float32)
  z64 = jnp.zeros((CH, 64), jnp.float32)

  gpre = _sc_gather(nf, gidx, 16, K_G).reshape(2, EP // 2, 32)
  ea16p, escal = _tc_preproc(gpre)
  na = _sc_scatter_add(ea16p.reshape(EP, 16), sidx, z16, 16, K_E,
                       S_NODE, ACC_N)
  mg = _sc_scatter_add(nf, bidx, z16, 16, K_B, S_G, ACC_G)
  mp = _sc_gather(mg, midx4, 16, K_M)[:NP]

  w_emb, b_emb = _tp_weights(params['emb'])
  w_emb = jnp.pad(w_emb, ((0, 0), (0, 1), (0, 0)))
  x, nat = _tc_embed(nf.reshape(NP // 2, 32), mp.reshape(NP // 2, 32),
                     na.reshape(NP // 2, 32), _bd(w_emb), _b2(b_emb))

  for lp in params['layers']:
    w1, b1 = _tp_weights(lp['m1'])
    wdp = jnp.concatenate([w1[:, 2 * H], w1[:, 2 * H + 1]], axis=0)
    w2, b2 = _tp_weights(lp['m2'])
    wu1, bu1 = _tp_weights(lp['u1'])
    wu2, bu2 = _tp_weights(lp['u2'])

    wdb, wsb = _bd(w1[:, :H]), _bd(w1[:, H:2 * H])
    wdpb, b1b, w2b, b2b = _bd1(wdp), _b2(b1), _bd(w2), _b2(b2)
    xf = x.reshape(NP, H)
    g3a = _sc_gather(xf, gidx_a, H, K_G // 2).reshape(2, EP // 4, 128)
    m2a = _tc_edge(g3a, escal, 0, wdb, wsb, wdpb, b1b, w2b, b2b)
    g3b = _sc_gather(xf, gidx_b, H, K_G // 2).reshape(2, EP // 4, 128)
    m2b = _tc_edge(g3b, escal, EP // 4 // (BE // 2), wdb, wsb, wdpb,
                   b1b, w2b, b2b)
    agg_a = _sc_scatter_add(m2a.reshape(EH, H), sidx_a, z64, H, K_E // 2,
                            S_NODE, ACC_N).reshape(NP // 2, 128)
    agg_b = _sc_scatter_add(m2b.reshape(EH, H), sidx_b, z64, H, K_E // 2,
                            S_NODE, ACC_N).reshape(NP // 2, 128)
    x = _tc_node(x, agg_a, agg_b, nat, _bd(wu1[:, :H]), _bd(wu1[:, H:]),
                 _b2(bu1), _bd(wu2), _b2(bu2))

  wo1, bo1 = _tp_weights(params['o1'])
  wo2, bo2 = _tp_weights(params['o2'])
  wo2 = jnp.pad(wo2, ((0, 0), (0, 0), (0, 125)))
  bo2 = jnp.pad(bo2, ((0, 0), (0, 125)))
  out = _tc_output(x, nat, nf.reshape(NP // 2, 32),
                   _bd(wo1), _b2(bo1), _bd(wo2), _b2(bo2))
  return out.reshape(NP, 128)[:N, :3]


def kernel(pos, vel, charges, params, edge_index, batch):
  return _run(pos, vel, charges, params, edge_index, batch)
